# Initial kernel scaffold; baseline (speedup 1.0000x reference)
import jax
import jax.numpy as jnp
from jax.experimental import pallas as pl


def kernel(node_feat, topic_feat, topic_probs):
    raise NotImplementedError("write your pallas kernel here")


# speedup vs baseline: 4.4063x; 4.4063x over previous
import functools

import jax
import jax.numpy as jnp
from jax import lax
from jax.experimental import pallas as pl
from jax.experimental.pallas import tpu as pltpu
from jax.experimental.pallas import tpu_sc as plsc

N = 16384
DF = 512
K = 128
NT = 64
M = 128
BLK = 2048


def _rowsum_xla(x):
    w = x.shape[-1]
    if w > 128:
        acc = x[:, 0:128]
        for c in range(1, w // 128):
            acc = acc + x[:, c * 128:(c + 1) * 128]
        x = acc
        w = 128
    s = x
    for g in range(1, w // 8):
        s = s + pltpu.roll(x, w - 8 * g, axis=1)
    for h in (4, 2, 1):
        s = s + pltpu.roll(s, w - h, axis=1)
    return s[:, :1]


def _tf_body(tp_ref, tfeat_ref, tf_ref):
    tp = tp_ref[...]
    mask = tp != 0.0
    x = jnp.where(mask, tp, -jnp.inf) / 0.01
    xmax = jnp.max(x, axis=1, keepdims=True)
    un = jnp.exp(x - xmax)
    probs = un / _rowsum_xla(un)
    tf_ref[...] = jnp.sum(tfeat_ref[...] * probs[:, :, None], axis=1)


_tf_call = pl.pallas_call(
    _tf_body,
    in_specs=[
        pl.BlockSpec((K, NT), lambda: (0, 0)),
        pl.BlockSpec((K, NT, DF), lambda: (0, 0, 0)),
    ],
    out_specs=pl.BlockSpec((K, DF), lambda: (0, 0)),
    out_shape=jax.ShapeDtypeStruct((K, DF), jnp.float32),
)


def _pd_body(node_ref, tf_ref, pt_ref, dt_ref):
    a = node_ref[...]
    tf = tf_ref[...]
    a2 = _rowsum_xla(a * a)
    b2 = _rowsum_xla(tf * tf).reshape(1, K)
    ab = lax.dot_general(a, tf, (((1,), (1,)), ((), ())),
                         preferred_element_type=jnp.float32)
    d2 = a2 + b2 - 2.0 * ab
    d = jnp.sqrt(jnp.maximum(d2, 1e-12))
    x = -1.0 * d
    xmax = jnp.max(x, axis=1, keepdims=True)
    un = jnp.exp(x - xmax)
    p = un / _rowsum_xla(un)
    pt_ref[...] = p.T
    dt_ref[...] = d.T


_pd_call = pl.pallas_call(
    _pd_body,
    grid=(N // BLK,),
    in_specs=[
        pl.BlockSpec((BLK, DF), lambda i: (i, 0)),
        pl.BlockSpec((K, DF), lambda i: (0, 0)),
    ],
    out_specs=[
        pl.BlockSpec((K, BLK), lambda i: (0, i)),
        pl.BlockSpec((K, BLK), lambda i: (0, i)),
    ],
    out_shape=[
        jax.ShapeDtypeStruct((K, N), jnp.float32),
        jax.ShapeDtypeStruct((K, N), jnp.float32),
    ],
)


def _lanes():
    return lax.iota(jnp.int32, 16)


def _shuf(x, c):
    return x.at[_lanes() ^ c].get(mode="promise_in_bounds")


def _allmin(x):
    for c in (1, 2, 4, 8):
        x = jnp.minimum(x, _shuf(x, c))
    return x


def _allmax(x):
    for c in (1, 2, 4, 8):
        x = jnp.maximum(x, _shuf(x, c))
    return x


def _bcast0(x):
    return x.at[jnp.zeros((16,), jnp.int32)].get(mode="promise_in_bounds")


def _sc_body(pt_hbm, dt_hbm, outt_hbm, cent_hbm,
             pv0, pv1, dv0, dv1, vm0, vm1, vm20, vm21, oi0, oi1, cvec):
    INF = jnp.float32(jnp.inf)
    BIG = jnp.int32(2**30)
    lanes = _lanes()
    wid = lax.axis_index("s") * 2 + lax.axis_index("c")
    cols = ((pv0, dv0, vm0, vm20, oi0), (pv1, dv1, vm1, vm21, oi1))

    def pair_body(j, _):
        ks = [wid * 4 + j * 2, wid * 4 + j * 2 + 1]
        for (pv, dv, _vm, _vm2, _oi), k in zip(cols, ks):
            pltpu.sync_copy(pt_hbm.at[k], pv.at[pl.ds(0, N)])
            pltpu.sync_copy(dt_hbm.at[k], dv.at[pl.ds(0, N)])

        def vm_body(w, _):
            for pv, dv, _vm, _vm2, _oi in cols:
                acc = jnp.full((16,), INF, jnp.float32)
                for t in range(16):
                    x = pv[pl.ds(w * 256 + t * 16, 16)]
                    acc = jnp.where(lanes == t, _allmin(x), acc)
                _vm[pl.ds(w * 16, 16)] = acc
            return 0

        lax.fori_loop(0, N // 256, vm_body, jnp.int32(0))

        for pv, dv, _vm, _vm2, _oi in cols:
            for w2 in range(N // 4096):
                acc = jnp.full((16,), INF, jnp.float32)
                for t in range(16):
                    x = _vm[pl.ds(w2 * 256 + t * 16, 16)]
                    acc = jnp.where(lanes == t, _allmin(x), acc)
                _vm2[pl.ds(w2 * 16, 16)] = acc

        def ext_body(i, carry):
            out = []
            for (pv, dv, _vm, _vm2, _oi), (bd, bc) in zip(cols, carry):
                bval = jnp.full((16,), INF, jnp.float32)
                bidx = jnp.full((16,), BIG, jnp.int32)
                for w2 in range(N // 4096):
                    v = _vm2[pl.ds(w2 * 16, 16)]
                    idxs = lanes + w2 * 16
                    take = (v < bval) | ((v == bval) & (idxs < bidx))
                    bval = jnp.where(take, v, bval)
                    bidx = jnp.where(take, idxs, bidx)
                mval = _allmin(bval)
                jsel = _allmin(jnp.where(bval == mval, bidx, BIG))
                j_s = jsel[0]
                vmv = _vm[pl.ds(j_s * 16, 16)]
                lane2 = _allmin(jnp.where(vmv == mval, lanes, BIG))
                e_s = (j_s * 16 + lane2)[0]
                sv = pv[pl.ds(e_s * 16, 16)]
                lane3 = _allmin(jnp.where(sv == mval, lanes, BIG))
                rowv = e_s * 16 + lane3
                row_s = rowv[0]
                cur = _oi[pl.ds(i, 16)]
                _oi[pl.ds(i, 16)] = jnp.where(lanes == 0, rowv, cur)
                dsc = dv[pl.ds(row_s, 16)][0]
                tk = dsc < bd
                bd = jnp.where(tk, dsc, bd)
                bc = jnp.where(tk, row_s, bc)
                sv2 = jnp.where(lanes == lane3, INF, sv)
                pv[pl.ds(e_s * 16, 16)] = sv2
                nm = _allmin(sv2)
                vcur = _vm[pl.ds(e_s, 16)]
                _vm[pl.ds(e_s, 16)] = jnp.where(lanes == 0, nm, vcur)
                nm2 = _allmin(jnp.where(lanes == lane2, nm, vmv))
                c2 = _vm2[pl.ds(j_s, 16)]
                _vm2[pl.ds(j_s, 16)] = jnp.where(lanes == 0, nm2, c2)
                out.append((bd, bc))
            return tuple(out)

        carry = lax.fori_loop(0, M, ext_body, ((INF, BIG), (INF, BIG)))
        for (pv, dv, _vm, _vm2, _oi), (bd, bc), k in zip(cols, carry, ks):
            pltpu.sync_copy(_oi.at[pl.ds(0, M)], outt_hbm.at[k])
            cvec[...] = jnp.full((16,), bc, jnp.int32)
            pltpu.sync_copy(cvec, cent_hbm.at[k])
        return 0

    lax.fori_loop(0, K // 64, pair_body, jnp.int32(0))


_sc_call = pl.kernel(
    _sc_body,
    out_type=(
        jax.ShapeDtypeStruct((K, M), jnp.int32),
        jax.ShapeDtypeStruct((K, 16), jnp.int32),
    ),
    mesh=plsc.VectorSubcoreMesh(core_axis_name="c", subcore_axis_name="s",
                                num_cores=2, num_subcores=16),
    scratch_types=[
        pltpu.VMEM((N + 16,), jnp.float32),
        pltpu.VMEM((N + 16,), jnp.float32),
        pltpu.VMEM((N + 16,), jnp.float32),
        pltpu.VMEM((N + 16,), jnp.float32),
        pltpu.VMEM((N // 16 + 16,), jnp.float32),
        pltpu.VMEM((N // 16 + 16,), jnp.float32),
        pltpu.VMEM((N // 256 + 16,), jnp.float32),
        pltpu.VMEM((N // 256 + 16,), jnp.float32),
        pltpu.VMEM((M + 16,), jnp.int32),
        pltpu.VMEM((M + 16,), jnp.int32),
        pltpu.VMEM((16,), jnp.int32),
    ],
)


def kernel(node_feat, topic_feat, topic_probs):
    tfm = _tf_call(topic_probs, topic_feat)
    pt, dt = _pd_call(node_feat, tfm)
    outt, centb = _sc_call(pt, dt)
    return outt.T, centb[:, 0]
